# baseline (speedup 1.0000x reference)
import functools

import jax
import jax.numpy as jnp
from jax import lax
from jax.experimental import pallas as pl
from jax.experimental.pallas import tpu as pltpu
from jax.experimental.pallas import tpu_sc as plsc

NC, NS, LANES = 2, 16, 16
NW = NC * NS
NUM_T = 4
NUM_CLASSES = 10
TAU = 10.0
BATCH = 2048
BCH = BATCH // LANES
CHUNK = 16


def _enc_body(x_ref, o_ref):
    x = x_ref[...]
    t = lax.broadcasted_iota(jnp.int32, (1, NUM_T, 1), 1).astype(jnp.float32)
    thr = (t + 1.0) / (NUM_T + 1.0)
    o_ref[...] = jnp.where(x[:, None, :] >= thr, 1.0, 0.0)


def _encode(xT):
    f = xT.shape[0]
    blk = 256
    return pl.pallas_call(
        _enc_body,
        grid=(f // blk,),
        in_specs=[pl.BlockSpec((blk, BATCH), lambda i: (i, 0))],
        out_specs=pl.BlockSpec((blk, NUM_T, BATCH), lambda i: (i, 0, 0)),
        out_shape=jax.ShapeDtypeStruct((f, NUM_T, BATCH), jnp.float32),
    )(xT)


def _coef_body(scale, l_ref, o_ref):
    om = 0.5 + 0.5 * jnp.sin(l_ref[...])
    w00, w01, w10, w11 = om[0:1], om[1:2], om[2:3], om[3:4]
    a = w00
    b = w10 - w00
    c = w01 - w00
    d = w00 - w01 - w10 + w11
    o_ref[...] = jnp.concatenate([a, b, c, d], axis=0) * scale


def _coefs(logitsT, scale):
    return pl.pallas_call(
        functools.partial(_coef_body, scale),
        out_shape=jax.ShapeDtypeStruct(logitsT.shape, jnp.float32),
    )(logitsT)


def _head_body(p_ref, o_ref):
    o_ref[...] = jnp.sum(p_ref[...], axis=0)


def _head_sum(parts):
    return pl.pallas_call(
        _head_body,
        out_shape=jax.ShapeDtypeStruct((NUM_CLASSES, BATCH), jnp.float32),
    )(parts)


def _prep(left, right, logitsT, w_pad, scale):
    w = left.shape[0]
    pad = w_pad - w
    zi = jnp.zeros((pad,), jnp.int32)
    lp = jnp.concatenate([left, zi])
    rp = jnp.concatenate([right, zi])
    cf = _coefs(logitsT, scale)
    cfp = jnp.concatenate([cf, jnp.zeros((4, pad), jnp.float32)], axis=1)
    return lp, rp, cfp


def _mesh():
    return plsc.VectorSubcoreMesh(core_axis_name="c", subcore_axis_name="s")


def _gate_layer(hin, left_p, right_p, coef_p, w_pad):
    gpt = w_pad // NW
    nch = gpt // CHUNK

    @functools.partial(
        pl.kernel,
        out_type=jax.ShapeDtypeStruct((w_pad, BATCH), jnp.float32),
        mesh=_mesh(),
        scratch_types=[
            pltpu.VMEM((CHUNK,), jnp.int32),
            pltpu.VMEM((CHUNK,), jnp.int32),
            pltpu.VMEM((4, gpt), jnp.float32),
            pltpu.VMEM((CHUNK, BATCH), jnp.float32),
            pltpu.VMEM((CHUNK, BATCH), jnp.float32),
            pltpu.VMEM((CHUNK, BATCH), jnp.float32),
            pltpu.SemaphoreType.DMA,
            pltpu.SemaphoreType.DMA,
        ],
    )
    def k(hin_h, left_h, right_h, coef_h, out_h,
          idxl, idxr, cf, lbuf, rbuf, obuf, seml, semr):
        wid = lax.axis_index("s") * NC + lax.axis_index("c")
        base = wid * gpt
        pltpu.sync_copy(coef_h.at[:, pl.ds(base, gpt)], cf)

        def chunk(ci, _):
            g0 = base + ci * CHUNK
            pltpu.sync_copy(left_h.at[pl.ds(g0, CHUNK)], idxl)
            pltpu.sync_copy(right_h.at[pl.ds(g0, CHUNK)], idxr)
            cpl = pltpu.async_copy(hin_h.at[idxl], lbuf, seml)
            cpr = pltpu.async_copy(hin_h.at[idxr], rbuf, semr)
            av = cf[0, pl.ds(ci * CHUNK, CHUNK)]
            bv = cf[1, pl.ds(ci * CHUNK, CHUNK)]
            cv = cf[2, pl.ds(ci * CHUNK, CHUNK)]
            dv = cf[3, pl.ds(ci * CHUNK, CHUNK)]
            cpl.wait()
            cpr.wait()
            for j in range(CHUNK):
                a, b, c, d = av[j], bv[j], cv[j], dv[j]

                def bstep(bi, _):
                    s = bi * LANES
                    l = lbuf[j, pl.ds(s, LANES)]
                    r = rbuf[j, pl.ds(s, LANES)]
                    obuf[j, pl.ds(s, LANES)] = (a + b * l) + r * (c + d * l)
                    return 0

                lax.fori_loop(0, BCH, bstep, 0, unroll=4)
            pltpu.sync_copy(obuf, out_h.at[pl.ds(g0, CHUNK)])
            return 0

        lax.fori_loop(0, nch, chunk, 0)

    return k(hin, left_p, right_p, coef_p)


def _gate_head_layer(hin, left_p, right_p, coef_p, w_pad):
    gpt = w_pad // NW
    nch = gpt // CHUNK

    @functools.partial(
        pl.kernel,
        out_type=jax.ShapeDtypeStruct((NW, NUM_CLASSES, BATCH), jnp.float32),
        mesh=_mesh(),
        scratch_types=[
            pltpu.VMEM((CHUNK,), jnp.int32),
            pltpu.VMEM((CHUNK,), jnp.int32),
            pltpu.VMEM((4, gpt), jnp.float32),
            pltpu.VMEM((CHUNK, BATCH), jnp.float32),
            pltpu.VMEM((CHUNK, BATCH), jnp.float32),
            pltpu.VMEM((NUM_CLASSES, BATCH), jnp.float32),
            pltpu.SemaphoreType.DMA,
            pltpu.SemaphoreType.DMA,
        ],
    )
    def k(hin_h, left_h, right_h, coef_h, out_h,
          idxl, idxr, cf, lbuf, rbuf, acc, seml, semr):
        wid = lax.axis_index("s") * NC + lax.axis_index("c")
        base = wid * gpt
        pltpu.sync_copy(coef_h.at[:, pl.ds(base, gpt)], cf)

        def zrow(i, _):
            def zcol(bi, _):
                acc[i, pl.ds(bi * LANES, LANES)] = jnp.zeros(
                    (LANES,), jnp.float32)
                return 0
            lax.fori_loop(0, BCH, zcol, 0, unroll=4)
            return 0

        lax.fori_loop(0, NUM_CLASSES, zrow, 0)

        def chunk(ci, _):
            g0 = base + ci * CHUNK
            pltpu.sync_copy(left_h.at[pl.ds(g0, CHUNK)], idxl)
            pltpu.sync_copy(right_h.at[pl.ds(g0, CHUNK)], idxr)
            cpl = pltpu.async_copy(hin_h.at[idxl], lbuf, seml)
            cpr = pltpu.async_copy(hin_h.at[idxr], rbuf, semr)
            av = cf[0, pl.ds(ci * CHUNK, CHUNK)]
            bv = cf[1, pl.ds(ci * CHUNK, CHUNK)]
            cv = cf[2, pl.ds(ci * CHUNK, CHUNK)]
            dv = cf[3, pl.ds(ci * CHUNK, CHUNK)]
            cpl.wait()
            cpr.wait()
            for j in range(CHUNK):
                a, b, c, d = av[j], bv[j], cv[j], dv[j]
                cls = lax.min((g0 + j) // 1600, NUM_CLASSES - 1)

                def bstep(bi, _):
                    s = bi * LANES
                    l = lbuf[j, pl.ds(s, LANES)]
                    r = rbuf[j, pl.ds(s, LANES)]
                    acc[cls, pl.ds(s, LANES)] = acc[cls, pl.ds(s, LANES)] + (
                        (a + b * l) + r * (c + d * l))
                    return 0

                lax.fori_loop(0, BCH, bstep, 0, unroll=4)
            return 0

        lax.fori_loop(0, nch, chunk, 0)
        pltpu.sync_copy(acc, out_h.at[wid])

    return k(hin, left_p, right_p, coef_p)


WP01 = 24576
WP2 = 16384


def kernel(x, left0, right0, logits0, left1, right1, logits1,
           left2, right2, logits2):
    xT = x.reshape(BATCH, -1).T
    h0 = _encode(xT).reshape(-1, BATCH)

    l0, r0, c0 = _prep(left0, right0, logits0.T, WP01, 1.0)
    l1, r1, c1 = _prep(left1, right1, logits1.T, WP01, 1.0)
    l2, r2, c2 = _prep(left2, right2, logits2.T, WP2, 1.0 / TAU)

    h1 = _gate_layer(h0, l0, r0, c0, WP01)
    h2 = _gate_layer(h1, l1, r1, c1, WP01)
    parts = _gate_head_layer(h2, l2, r2, c2, WP2)
    return _head_sum(parts).T


# speedup vs baseline: 1.9468x; 1.9468x over previous
import functools

import jax
import jax.numpy as jnp
from jax import lax
from jax.experimental import pallas as pl
from jax.experimental.pallas import tpu as pltpu
from jax.experimental.pallas import tpu_sc as plsc

NC, NS, LANES = 2, 16, 16
NW = NC * NS
NUM_T = 4
NUM_CLASSES = 10
TAU = 10.0
BATCH = 2048
BCH = BATCH // LANES
CHUNK = 8


def _enc_body(x_ref, o_ref):
    x = x_ref[...]
    t = lax.broadcasted_iota(jnp.int32, (1, NUM_T, 1), 1).astype(jnp.float32)
    thr = (t + 1.0) / (NUM_T + 1.0)
    o_ref[...] = jnp.where(x[:, None, :] >= thr, 1.0, 0.0)


def _encode(xT):
    f = xT.shape[0]
    blk = 256
    return pl.pallas_call(
        _enc_body,
        grid=(f // blk,),
        in_specs=[pl.BlockSpec((blk, BATCH), lambda i: (i, 0))],
        out_specs=pl.BlockSpec((blk, NUM_T, BATCH), lambda i: (i, 0, 0)),
        out_shape=jax.ShapeDtypeStruct((f, NUM_T, BATCH), jnp.float32),
    )(xT)


def _coef_body(scale, l_ref, o_ref):
    om = 0.5 + 0.5 * jnp.sin(l_ref[...])
    w00, w01, w10, w11 = om[0:1], om[1:2], om[2:3], om[3:4]
    a = w00
    b = w10 - w00
    c = w01 - w00
    d = w00 - w01 - w10 + w11
    o_ref[...] = jnp.concatenate([a, b, c, d], axis=0) * scale


def _coefs(logitsT, scale):
    return pl.pallas_call(
        functools.partial(_coef_body, scale),
        out_shape=jax.ShapeDtypeStruct(logitsT.shape, jnp.float32),
    )(logitsT)


def _head_body(p_ref, o_ref):
    o_ref[...] = jnp.sum(p_ref[...], axis=0)


def _head_sum(parts):
    return pl.pallas_call(
        _head_body,
        out_shape=jax.ShapeDtypeStruct((NUM_CLASSES, BATCH), jnp.float32),
    )(parts)


def _prep(left, right, logitsT, w_pad, scale):
    w = left.shape[0]
    pad = w_pad - w
    zi = jnp.zeros((pad,), jnp.int32)
    lp = jnp.concatenate([left, zi])
    rp = jnp.concatenate([right, zi])
    cf = _coefs(logitsT, scale)
    cfp = jnp.concatenate([cf, jnp.zeros((4, pad), jnp.float32)], axis=1)
    return lp, rp, cfp


def _mesh():
    return plsc.VectorSubcoreMesh(core_axis_name="c", subcore_axis_name="s")


def _pair_coefs(cf, g16):
    av = cf[0, pl.ds(g16, 16)]
    bv = cf[1, pl.ds(g16, 16)]
    cv = cf[2, pl.ds(g16, 16)]
    dv = cf[3, pl.ds(g16, 16)]
    return [(av[j], bv[j], cv[j], dv[j]) for j in range(16)]


def _gate_layer(hin, left_p, right_p, coef_p, w_pad):
    gpt = w_pad // NW
    nch = gpt // CHUNK

    @functools.partial(
        pl.kernel,
        out_type=jax.ShapeDtypeStruct((w_pad, BATCH), jnp.float32),
        mesh=_mesh(),
        scratch_types=[
            pltpu.VMEM((gpt,), jnp.int32),
            pltpu.VMEM((gpt,), jnp.int32),
            pltpu.VMEM((4, gpt), jnp.float32),
            pltpu.VMEM((CHUNK, BATCH), jnp.float32),
            pltpu.VMEM((CHUNK, BATCH), jnp.float32),
            pltpu.VMEM((CHUNK, BATCH), jnp.float32),
            pltpu.VMEM((CHUNK, BATCH), jnp.float32),
            pltpu.VMEM((CHUNK, BATCH), jnp.float32),
            pltpu.SemaphoreType.DMA,
            pltpu.SemaphoreType.DMA,
            pltpu.SemaphoreType.DMA,
            pltpu.SemaphoreType.DMA,
        ],
    )
    def k(hin_h, left_h, right_h, coef_h, out_h,
          idxl, idxr, cf, lb0, rb0, lb1, rb1, obuf,
          sl0, sr0, sl1, sr1):
        wid = lax.axis_index("s") * NC + lax.axis_index("c")
        base = wid * gpt
        pltpu.sync_copy(left_h.at[pl.ds(base, gpt)], idxl)
        pltpu.sync_copy(right_h.at[pl.ds(base, gpt)], idxr)
        pltpu.sync_copy(coef_h.at[:, pl.ds(base, gpt)], cf)

        def gather(c, lb, rb, sl, sr):
            g = c * CHUNK
            return (pltpu.async_copy(hin_h.at[idxl.at[pl.ds(g, CHUNK)]],
                                     lb, sl),
                    pltpu.async_copy(hin_h.at[idxr.at[pl.ds(g, CHUNK)]],
                                     rb, sr))

        def compute_store(c, sc, lb, rb):
            g = c * CHUNK

            def bstep(bi, _):
                s = bi * LANES
                for j, (a, b, cc, d) in enumerate(sc):
                    l = lb[j, pl.ds(s, LANES)]
                    r = rb[j, pl.ds(s, LANES)]
                    obuf[j, pl.ds(s, LANES)] = (a + b * l) + r * (cc + d * l)
                return 0

            lax.fori_loop(0, BCH, bstep, 0, unroll=2)
            pltpu.sync_copy(obuf, out_h.at[pl.ds(base + g, CHUNK)])

        gather(0, lb0, rb0, sl0, sr0)

        def body(ci2, _):
            c0 = ci2 * 2
            sc = _pair_coefs(cf, c0 * CHUNK)
            w1 = gather(c0 + 1, lb1, rb1, sl1, sr1)
            pltpu.make_async_copy(hin_h.at[idxl.at[pl.ds(0, CHUNK)]],
                                  lb0, sl0).wait()
            pltpu.make_async_copy(hin_h.at[idxr.at[pl.ds(0, CHUNK)]],
                                  rb0, sr0).wait()
            compute_store(c0, sc[:CHUNK], lb0, rb0)

            @pl.when(ci2 * 2 + 2 < nch)
            def _():
                gather(c0 + 2, lb0, rb0, sl0, sr0)

            w1[0].wait()
            w1[1].wait()
            compute_store(c0 + 1, sc[CHUNK:], lb1, rb1)
            return 0

        lax.fori_loop(0, nch // 2, body, 0)

    return k(hin, left_p, right_p, coef_p)


def _gate_head_layer(hin, left_p, right_p, coef_p, w_pad):
    gpt = w_pad // NW
    nch = gpt // CHUNK

    @functools.partial(
        pl.kernel,
        out_type=jax.ShapeDtypeStruct((NW, NUM_CLASSES, BATCH), jnp.float32),
        mesh=_mesh(),
        scratch_types=[
            pltpu.VMEM((gpt,), jnp.int32),
            pltpu.VMEM((gpt,), jnp.int32),
            pltpu.VMEM((4, gpt), jnp.float32),
            pltpu.VMEM((CHUNK, BATCH), jnp.float32),
            pltpu.VMEM((CHUNK, BATCH), jnp.float32),
            pltpu.VMEM((CHUNK, BATCH), jnp.float32),
            pltpu.VMEM((CHUNK, BATCH), jnp.float32),
            pltpu.VMEM((NUM_CLASSES, BATCH), jnp.float32),
            pltpu.SemaphoreType.DMA,
            pltpu.SemaphoreType.DMA,
            pltpu.SemaphoreType.DMA,
            pltpu.SemaphoreType.DMA,
        ],
    )
    def k(hin_h, left_h, right_h, coef_h, out_h,
          idxl, idxr, cf, lb0, rb0, lb1, rb1, acc,
          sl0, sr0, sl1, sr1):
        wid = lax.axis_index("s") * NC + lax.axis_index("c")
        base = wid * gpt
        pltpu.sync_copy(left_h.at[pl.ds(base, gpt)], idxl)
        pltpu.sync_copy(right_h.at[pl.ds(base, gpt)], idxr)
        pltpu.sync_copy(coef_h.at[:, pl.ds(base, gpt)], cf)

        def zrow(i, _):
            def zcol(bi, _):
                acc[i, pl.ds(bi * LANES, LANES)] = jnp.zeros(
                    (LANES,), jnp.float32)
                return 0
            lax.fori_loop(0, BCH, zcol, 0, unroll=4)
            return 0

        lax.fori_loop(0, NUM_CLASSES, zrow, 0)

        def gather(c, lb, rb, sl, sr):
            g = c * CHUNK
            return (pltpu.async_copy(hin_h.at[idxl.at[pl.ds(g, CHUNK)]],
                                     lb, sl),
                    pltpu.async_copy(hin_h.at[idxr.at[pl.ds(g, CHUNK)]],
                                     rb, sr))

        def compute_acc(c, sc, lb, rb):
            g = c * CHUNK
            cls = lax.min((base + g) // 1600, NUM_CLASSES - 1)

            def bstep(bi, _):
                s = bi * LANES
                tot = acc[cls, pl.ds(s, LANES)]
                for j, (a, b, cc, d) in enumerate(sc):
                    l = lb[j, pl.ds(s, LANES)]
                    r = rb[j, pl.ds(s, LANES)]
                    tot = tot + (a + b * l) + r * (cc + d * l)
                acc[cls, pl.ds(s, LANES)] = tot
                return 0

            lax.fori_loop(0, BCH, bstep, 0, unroll=2)

        gather(0, lb0, rb0, sl0, sr0)

        def body(ci2, _):
            c0 = ci2 * 2
            sc = _pair_coefs(cf, c0 * CHUNK)
            w1 = gather(c0 + 1, lb1, rb1, sl1, sr1)
            pltpu.make_async_copy(hin_h.at[idxl.at[pl.ds(0, CHUNK)]],
                                  lb0, sl0).wait()
            pltpu.make_async_copy(hin_h.at[idxr.at[pl.ds(0, CHUNK)]],
                                  rb0, sr0).wait()
            compute_acc(c0, sc[:CHUNK], lb0, rb0)

            @pl.when(ci2 * 2 + 2 < nch)
            def _():
                gather(c0 + 2, lb0, rb0, sl0, sr0)

            w1[0].wait()
            w1[1].wait()
            compute_acc(c0 + 1, sc[CHUNK:], lb1, rb1)
            return 0

        lax.fori_loop(0, nch // 2, body, 0)
        pltpu.sync_copy(acc, out_h.at[wid])

    return k(hin, left_p, right_p, coef_p)


WP01 = 24576
WP2 = 16384


def kernel(x, left0, right0, logits0, left1, right1, logits1,
           left2, right2, logits2):
    xT = x.reshape(BATCH, -1).T
    h0 = _encode(xT).reshape(-1, BATCH)

    l0, r0, c0 = _prep(left0, right0, logits0.T, WP01, 1.0)
    l1, r1, c1 = _prep(left1, right1, logits1.T, WP01, 1.0)
    l2, r2, c2 = _prep(left2, right2, logits2.T, WP2, 1.0 / TAU)

    h1 = _gate_layer(h0, l0, r0, c0, WP01)
    h2 = _gate_layer(h1, l1, r1, c1, WP01)
    parts = _gate_head_layer(h2, l2, r2, c2, WP2)
    return _head_sum(parts).T
